# Initial kernel scaffold; baseline (speedup 1.0000x reference)
import jax
import jax.numpy as jnp
from jax.experimental import pallas as pl


def kernel(x, edge_index, batch, W1, a_src1, a_dst1, b1, W2, a_src2, a_dst2, b2, W3, b3, W4, b4, W5, b5, W6, b6, Wf, bf):
    raise NotImplementedError("write your pallas kernel here")


# speedup vs baseline: 1.9670x; 1.9670x over previous
import functools
import math

import jax
import jax.numpy as jnp
from jax import lax
from jax.experimental import pallas as pl
from jax.experimental.pallas import tpu as pltpu
from jax.experimental.pallas import tpu_sc as plsc

_R = 256
_NB_ROWS = 1024
_D = 48
_NEG = -1e30


def _gat_table_kern(x_ref, w_ref, as_ref, ad_ref, tab_ref, d_ref):
    h = jnp.dot(x_ref[...], w_ref[...], preferred_element_type=jnp.float32)
    s = jnp.sum(h * as_ref[...], axis=1, keepdims=True)
    d = jnp.sum(h * ad_ref[...], axis=1, keepdims=True)
    pad = jnp.zeros((h.shape[0], _D - h.shape[1] - 1), jnp.float32)
    tab_ref[...] = jnp.concatenate([h, s, pad], axis=1)
    d_ref[...] = d


def _gcn_table_kern(x_ref, w_ref, dinv_ref, tab_ref):
    h = jnp.dot(x_ref[...], w_ref[...], preferred_element_type=jnp.float32)
    pad = jnp.zeros((h.shape[0], _D - h.shape[1] - 1), jnp.float32)
    tab_ref[...] = jnp.concatenate([h, dinv_ref[...], pad], axis=1)


def _onehot(dl, k):
    ids = lax.broadcasted_iota(jnp.int32, (dl.shape[0], _R), 1)
    return dl[:, None] == ids


def _deg_kern(dl_ref, dinv_ref):
    pf = _onehot(dl_ref[0, 0], _R).astype(jnp.float32)
    deg = jnp.sum(pf, axis=0)[:, None]
    dinv_ref[...] = jnp.where(deg > 0, lax.rsqrt(jnp.maximum(deg, 1.0)), 0.0)


def _gat_bucket_kern(g_ref, dl_ref, d_ref, b_ref, out_ref):
    g = g_ref[0]
    h_src = g[:, :40]
    s_src = g[:, 40]
    pb = _onehot(dl_ref[0, 0], _R)
    pf = pb.astype(jnp.float32)
    d_exp = jnp.dot(pf, d_ref[...], preferred_element_type=jnp.float32)[:, 0]
    alpha = s_src + d_exp
    alpha = jnp.where(alpha > 0, alpha, 0.2 * alpha)
    masked = jnp.where(pb, alpha[:, None], _NEG)
    amax = jnp.max(masked, axis=0)
    amax_exp = jnp.dot(pf, amax[:, None],
                       preferred_element_type=jnp.float32)[:, 0]
    ev = jnp.exp(alpha - amax_exp)
    den = jax.lax.dot_general(pf, ev[:, None], (((0,), (0,)), ((), ())),
                              preferred_element_type=jnp.float32)[:, 0]
    den_exp = jnp.dot(pf, den[:, None],
                      preferred_element_type=jnp.float32)[:, 0]
    coef = ev / (den_exp + 1e-16)
    out = jax.lax.dot_general(pf * coef[:, None], h_src,
                              (((0,), (0,)), ((), ())),
                              preferred_element_type=jnp.float32)
    out_ref[...] = jnp.maximum(out + b_ref[...], 0.0)


def _gcn_bucket_kern(g_ref, dl_ref, dinv_ref, b_ref, out_ref):
    g = g_ref[0]
    h_src = g[:, :40]
    dinv_src = g[:, 40]
    pf = _onehot(dl_ref[0, 0], _R).astype(jnp.float32)
    dinv_exp = jnp.dot(pf, dinv_ref[...],
                       preferred_element_type=jnp.float32)[:, 0]
    norm = dinv_src * dinv_exp
    out = jax.lax.dot_general(pf * norm[:, None], h_src,
                              (((0,), (0,)), ((), ())),
                              preferred_element_type=jnp.float32)
    out_ref[...] = jnp.maximum(out + b_ref[...], 0.0)


def _pool_kern(x_ref, batch_ref, out_ref):
    i = pl.program_id(0)

    @pl.when(i == 0)
    def _():
        out_ref[...] = jnp.full(out_ref.shape, _NEG, jnp.float32)

    bcol = batch_ref[...]
    xv = x_ref[...]
    for b in range(64):
        mb = jnp.max(jnp.where(bcol == b, xv, _NEG), axis=0)
        out_ref[b, :] = jnp.maximum(out_ref[b, :], mb)


def _head_kern(g_ref, wf_ref, bf_ref, out_ref):
    logits = jnp.dot(g_ref[...], wf_ref[...],
                     preferred_element_type=jnp.float32) + bf_ref[...]
    m = jnp.max(logits, axis=1, keepdims=True)
    lse = jnp.log(jnp.sum(jnp.exp(logits - m), axis=1, keepdims=True)) + m
    out_ref[...] = logits - lse


def _make_sc_gather(n_rows, e_tot):
    try:
        info = plsc.get_sparse_core_info()
        nc, ns = info.num_cores, info.num_subcores
    except ValueError:
        nc, ns = 2, 16
    nw = nc * ns
    per_w = e_tot // nw
    chunks = per_w // 128
    mesh = plsc.VectorSubcoreMesh(core_axis_name="c", subcore_axis_name="s",
                                  num_cores=nc, num_subcores=ns)

    @functools.partial(
        pl.kernel, mesh=mesh,
        compiler_params=pltpu.CompilerParams(use_tc_tiling_on_sc=False),
        out_type=jax.ShapeDtypeStruct((e_tot, _D), jnp.float32),
        scratch_types=[
            pltpu.VMEM((128,), jnp.int32),
            pltpu.VMEM((128, _D), jnp.float32),
            pltpu.SemaphoreType.DMA,
        ],
    )
    def k(tab_hbm, idx_hbm, out_hbm, idx_v, rows_v, sem):
        wid = lax.axis_index("s") * nc + lax.axis_index("c")

        def body(t, _):
            base = wid * per_w + t * 128
            pltpu.sync_copy(idx_hbm.at[pl.ds(base, 128)], idx_v)
            pltpu.async_copy(tab_hbm.at[idx_v], rows_v, sem).wait()
            pltpu.sync_copy(rows_v, out_hbm.at[pl.ds(base, 128)])
            return _

        lax.fori_loop(0, chunks, body, None)

    return k


def kernel(x, edge_index, batch, W1, a_src1, a_dst1, b1, W2, a_src2, a_dst2,
           b2, W3, b3, W4, b4, W5, b5, W6, b6, Wf, bf):
    n = x.shape[0]
    n_pad = ((n + _R - 1) // _R) * _R
    nb = n_pad // _R

    loop = jnp.arange(n, dtype=edge_index.dtype)
    src = jnp.concatenate([edge_index[0], loop])
    dst = jnp.concatenate([edge_index[1], loop])
    e_all = src.shape[0]
    order = jnp.argsort(dst)
    src_s = src[order]
    dst_s = dst[order]

    mean = e_all * _R / n
    kmax = int(mean + 12.0 * math.sqrt(mean) + 64.0)
    kmax = ((kmax + 127) // 128) * 128

    bucket = dst_s // _R
    bstart = jnp.searchsorted(dst_s, jnp.arange(nb, dtype=dst_s.dtype) * _R)
    rank = jnp.arange(e_all, dtype=jnp.int32) - bstart[bucket].astype(jnp.int32)
    flat_pos = bucket.astype(jnp.int32) * kmax + rank
    e_tot = nb * kmax
    e_tot_pad = ((e_tot + 4095) // 4096) * 4096
    src_pad = jnp.full((e_tot_pad,), n_pad, jnp.int32).at[flat_pos].set(
        src_s.astype(jnp.int32), mode="drop")
    dl_pad = jnp.full((e_tot,), _R, jnp.int32).at[flat_pos].set(
        (dst_s - bucket * _R).astype(jnp.int32), mode="drop")
    dl_pad = dl_pad.reshape(nb, 1, kmax)

    x_pad = jnp.pad(x, ((0, n_pad - n), (0, 0)))
    batch_pad = jnp.pad(batch, (0, n_pad - n),
                        constant_values=64).reshape(n_pad, 1)

    row_blk = _NB_ROWS if n_pad % _NB_ROWS == 0 else _R

    sc_gather = _make_sc_gather(n_pad + 8, e_tot_pad)

    def table_gat(xx, w, a_s, a_d):
        f_in = xx.shape[1]
        return pl.pallas_call(
            _gat_table_kern,
            grid=(n_pad // row_blk,),
            in_specs=[
                pl.BlockSpec((row_blk, f_in), lambda i: (i, 0)),
                pl.BlockSpec((f_in, 40), lambda i: (0, 0)),
                pl.BlockSpec((1, 40), lambda i: (0, 0)),
                pl.BlockSpec((1, 40), lambda i: (0, 0)),
            ],
            out_specs=[
                pl.BlockSpec((row_blk, _D), lambda i: (i, 0)),
                pl.BlockSpec((row_blk, 1), lambda i: (i, 0)),
            ],
            out_shape=[
                jax.ShapeDtypeStruct((n_pad, _D), jnp.float32),
                jax.ShapeDtypeStruct((n_pad, 1), jnp.float32),
            ],
        )(xx, w, a_s.reshape(1, 40), a_d.reshape(1, 40))

    def table_gcn(xx, w, dinv):
        f_in = xx.shape[1]
        return pl.pallas_call(
            _gcn_table_kern,
            grid=(n_pad // row_blk,),
            in_specs=[
                pl.BlockSpec((row_blk, f_in), lambda i: (i, 0)),
                pl.BlockSpec((f_in, 40), lambda i: (0, 0)),
                pl.BlockSpec((row_blk, 1), lambda i: (i, 0)),
            ],
            out_specs=pl.BlockSpec((row_blk, _D), lambda i: (i, 0)),
            out_shape=jax.ShapeDtypeStruct((n_pad, _D), jnp.float32),
        )(xx, w, dinv)

    def gather(tab):
        tab8 = jnp.pad(tab, ((0, 8), (0, 0)))
        g = sc_gather(tab8, src_pad)
        return g[:e_tot].reshape(nb, kmax, _D)

    def bucket_gat(g, d, b):
        return pl.pallas_call(
            _gat_bucket_kern,
            grid=(nb,),
            in_specs=[
                pl.BlockSpec((1, kmax, _D), lambda j: (j, 0, 0)),
                pl.BlockSpec((1, 1, kmax), lambda j: (j, 0, 0)),
                pl.BlockSpec((_R, 1), lambda j: (j, 0)),
                pl.BlockSpec((1, 40), lambda j: (0, 0)),
            ],
            out_specs=pl.BlockSpec((_R, 40), lambda j: (j, 0)),
            out_shape=jax.ShapeDtypeStruct((n_pad, 40), jnp.float32),
        )(g, dl_pad, d, b.reshape(1, 40))

    def bucket_gcn(g, dinv, b):
        return pl.pallas_call(
            _gcn_bucket_kern,
            grid=(nb,),
            in_specs=[
                pl.BlockSpec((1, kmax, _D), lambda j: (j, 0, 0)),
                pl.BlockSpec((1, 1, kmax), lambda j: (j, 0, 0)),
                pl.BlockSpec((_R, 1), lambda j: (j, 0)),
                pl.BlockSpec((1, 40), lambda j: (0, 0)),
            ],
            out_specs=pl.BlockSpec((_R, 40), lambda j: (j, 0)),
            out_shape=jax.ShapeDtypeStruct((n_pad, 40), jnp.float32),
        )(g, dl_pad, dinv, b.reshape(1, 40))

    dinv = pl.pallas_call(
        _deg_kern,
        grid=(nb,),
        in_specs=[pl.BlockSpec((1, 1, kmax), lambda j: (j, 0, 0))],
        out_specs=pl.BlockSpec((_R, 1), lambda j: (j, 0)),
        out_shape=jax.ShapeDtypeStruct((n_pad, 1), jnp.float32),
    )(dl_pad)

    tab, d1 = table_gat(x_pad, W1, a_src1, a_dst1)
    xx = bucket_gat(gather(tab), d1, b1)
    tab, d2 = table_gat(xx, W2, a_src2, a_dst2)
    xx = bucket_gat(gather(tab), d2, b2)
    for w, b in ((W3, b3), (W4, b4), (W5, b5), (W6, b6)):
        tab = table_gcn(xx, w, dinv)
        xx = bucket_gcn(gather(tab), dinv, b)

    g = pl.pallas_call(
        _pool_kern,
        grid=(n_pad // row_blk,),
        in_specs=[
            pl.BlockSpec((row_blk, 40), lambda i: (i, 0)),
            pl.BlockSpec((row_blk, 1), lambda i: (i, 0)),
        ],
        out_specs=pl.BlockSpec((64, 40), lambda i: (0, 0)),
        out_shape=jax.ShapeDtypeStruct((64, 40), jnp.float32),
    )(xx, batch_pad)

    return pl.pallas_call(
        _head_kern,
        in_specs=[
            pl.BlockSpec((64, 40), lambda: (0, 0)),
            pl.BlockSpec((40, 3), lambda: (0, 0)),
            pl.BlockSpec((1, 3), lambda: (0, 0)),
        ],
        out_specs=pl.BlockSpec((64, 3), lambda: (0, 0)),
        out_shape=jax.ShapeDtypeStruct((64, 3), jnp.float32),
    )(g, Wf, bf.reshape(1, 3))
